# baseline (speedup 1.0000x reference)
import functools

import jax
import jax.numpy as jnp
from jax import lax
from jax.experimental import pallas as pl
from jax.experimental.pallas import tpu as pltpu
from jax.experimental.pallas import tpu_sc as plsc

_TM = 2048


_WIN_EDGES = (0, 2736, 5472, 8192)
_CHUNKS = ((0, 1368), (1368, 2736), (2736, 4104), (4104, 5472),
           (5472, 6840), (6840, 8192))


def _argmin_body(z_ref, cb_ref, zsq_ref, csq_ref, idx_ref):
    z2 = z_ref[...] + z_ref[...]
    zsq = zsq_ref[...]
    tm = z2.shape[0]
    big = jnp.float32(2.0**30)
    col = lax.broadcasted_iota(jnp.int32, (tm, 1368), 1).astype(jnp.float32)

    acc_v = acc_i = None
    wv = wi = None
    for ch, (r0, r1) in enumerate(_CHUNKS):
        w = r1 - r0
        zc2 = lax.dot_general(
            z2, cb_ref[r0:r1, :], (((1,), (1,)), ((), ())),
            preferred_element_type=jnp.float32)
        d = (zsq - zc2) + csq_ref[:, r0:r1]
        lm = jnp.min(d, axis=1, keepdims=True)
        li = jnp.min(jnp.where(d == lm, col[:, :w], big),
                     axis=1, keepdims=True) + jnp.float32(r0)
        if ch % 2 == 0:
            wv, wi = lm, li
        else:
            better = lm < wv
            wi = jnp.where(better, li, wi)
            wv = jnp.where(better, lm, wv)
            if acc_v is None:
                acc_v, acc_i = wv, wi
            else:
                better = wv < acc_v
                acc_i = jnp.where(better, wi, acc_i)
                acc_v = jnp.where(better, wv, acc_v)
            acc_v = acc_v.astype(jnp.bfloat16).astype(jnp.float32)
    idx_ref[...] = acc_i.astype(jnp.int32)


def _nearest_idx_call(z, cb, zsq, csq, tm):
    n, c = z.shape
    k = cb.shape[0]
    grid = (n // tm,)
    return pl.pallas_call(
        _argmin_body,
        grid=grid,
        in_specs=[
            pl.BlockSpec((tm, c), lambda m: (m, 0)),
            pl.BlockSpec((k, c), lambda m: (0, 0)),
            pl.BlockSpec((tm, 1), lambda m: (m, 0)),
            pl.BlockSpec((1, k), lambda m: (0, 0)),
        ],
        out_specs=pl.BlockSpec((tm, 1), lambda m: (m, 0)),
        out_shape=jax.ShapeDtypeStruct((n, 1), jnp.int32),
        compiler_params=pltpu.CompilerParams(
            dimension_semantics=("arbitrary",)),
    )(z, cb, zsq, csq)


def _gather_rows_call(table, idx_flat):
    n = idx_flat.shape[0]
    d = table.shape[1]
    nc, ns = 2, 16
    nw = nc * ns
    chunk = 128
    per_w = n // nw
    n_chunks = per_w // chunk
    mesh = plsc.VectorSubcoreMesh(core_axis_name="c", subcore_axis_name="s")

    @functools.partial(
        pl.kernel,
        out_type=jax.ShapeDtypeStruct((n, d), jnp.float32),
        mesh=mesh,
        scratch_types=[
            pltpu.VMEM((2, chunk), jnp.int32),
            pltpu.VMEM((2, chunk, d), jnp.float32),
            pltpu.SemaphoreType.DMA,
            pltpu.SemaphoreType.DMA,
        ],
    )
    def gather_k(idx_hbm, table_hbm, out_hbm, idx_v, rows_v, sem0, sem1):
        wid = lax.axis_index("s") * nc + lax.axis_index("c")
        base = wid * per_w
        sems = (sem0, sem1)
        pltpu.sync_copy(idx_hbm.at[pl.ds(base, chunk)], idx_v.at[0])
        cp0 = pltpu.async_copy(table_hbm.at[idx_v.at[0]], rows_v.at[0], sems[0])
        copies = [cp0, None]
        for c in range(n_chunks):
            cur = c % 2
            nxt = (c + 1) % 2
            if c + 1 < n_chunks:
                off = base + (c + 1) * chunk
                pltpu.sync_copy(idx_hbm.at[pl.ds(off, chunk)], idx_v.at[nxt])
                copies[nxt] = pltpu.async_copy(
                    table_hbm.at[idx_v.at[nxt]], rows_v.at[nxt], sems[nxt])
            copies[cur].wait()
            pltpu.sync_copy(rows_v.at[cur],
                            out_hbm.at[pl.ds(base + c * chunk, chunk)])

    return gather_k(idx_flat, table)


def kernel(z_e, codebook):
    b, c, h, w = z_e.shape
    z = jnp.transpose(z_e, (0, 2, 3, 1)).reshape(-1, c)
    zsq = jnp.sum(z * z, axis=1, keepdims=True)
    csq = jnp.sum(codebook * codebook, axis=1)[None, :]
    n = z.shape[0]
    half = n // 2
    idx_a = _nearest_idx_call(z[:half], codebook, zsq[:half], csq, _TM)
    zq_a = _gather_rows_call(codebook, idx_a.reshape(-1))
    idx_b = _nearest_idx_call(z[half:], codebook, zsq[half:], csq, _TM)
    zq_b = _gather_rows_call(codebook, idx_b.reshape(-1))
    idx = jnp.concatenate([idx_a, idx_b], axis=0).reshape(-1)
    zq_flat = jnp.concatenate([zq_a, zq_b], axis=0)
    z_q = jnp.transpose(zq_flat.reshape(b, h, w, c), (0, 3, 1, 2))
    return z_q, idx.reshape(b, h, w)


# speedup vs baseline: 1.1155x; 1.1155x over previous
import functools

import jax
import jax.numpy as jnp
from jax import lax
from jax.experimental import pallas as pl
from jax.experimental.pallas import tpu as pltpu
from jax.experimental.pallas import tpu_sc as plsc

_TM = 2048


_WIN_EDGES = (0, 2736, 5472, 8192)
_CHUNKS = ((0, 1368), (1368, 2736), (2736, 4104), (4104, 5472),
           (5472, 6840), (6840, 8192))


def _argmin_body(z_ref, cb_ref, zsq_ref, csq_ref, idx_ref):
    zb = z_ref[...]
    z = jnp.transpose(zb, (0, 2, 1)).reshape(-1, zb.shape[1])
    z2 = z + z
    zsq = zsq_ref[...]
    tm = z2.shape[0]
    big = jnp.float32(2.0**30)
    col = lax.broadcasted_iota(jnp.int32, (tm, 1368), 1).astype(jnp.float32)

    acc_v = acc_i = None
    wv = wi = None
    for ch, (r0, r1) in enumerate(_CHUNKS):
        w = r1 - r0
        zc2 = lax.dot_general(
            z2, cb_ref[r0:r1, :], (((1,), (1,)), ((), ())),
            preferred_element_type=jnp.float32)
        d = (zsq - zc2) + csq_ref[:, r0:r1]
        lm = jnp.min(d, axis=1, keepdims=True)
        li = jnp.min(jnp.where(d == lm, col[:, :w], big),
                     axis=1, keepdims=True) + jnp.float32(r0)
        if ch % 2 == 0:
            wv, wi = lm, li
        else:
            better = lm < wv
            wi = jnp.where(better, li, wi)
            wv = jnp.where(better, lm, wv)
            if acc_v is None:
                acc_v, acc_i = wv, wi
            else:
                better = wv < acc_v
                acc_i = jnp.where(better, wi, acc_i)
                acc_v = jnp.where(better, wv, acc_v)
            acc_v = acc_v.astype(jnp.bfloat16).astype(jnp.float32)
    idx_ref[...] = acc_i.astype(jnp.int32)


def _nearest_idx_call(z3, cb, zsq, csq, tm):
    nb, c, hw = z3.shape
    n = nb * hw
    bpt = tm // hw
    k = cb.shape[0]
    grid = (n // tm,)
    return pl.pallas_call(
        _argmin_body,
        grid=grid,
        in_specs=[
            pl.BlockSpec((bpt, c, hw), lambda m: (m, 0, 0)),
            pl.BlockSpec((k, c), lambda m: (0, 0)),
            pl.BlockSpec((tm, 1), lambda m: (m, 0)),
            pl.BlockSpec((1, k), lambda m: (0, 0)),
        ],
        out_specs=pl.BlockSpec((tm, 1), lambda m: (m, 0)),
        out_shape=jax.ShapeDtypeStruct((n, 1), jnp.int32),
        compiler_params=pltpu.CompilerParams(
            dimension_semantics=("arbitrary",)),
    )(z3, cb, zsq, csq)


def _gather_rows_call(table, idx_flat):
    n = idx_flat.shape[0]
    d = table.shape[1]
    nc, ns = 2, 16
    nw = nc * ns
    chunk = 128
    per_w = n // nw
    n_chunks = per_w // chunk
    mesh = plsc.VectorSubcoreMesh(core_axis_name="c", subcore_axis_name="s")

    @functools.partial(
        pl.kernel,
        out_type=jax.ShapeDtypeStruct((n, d), jnp.float32),
        mesh=mesh,
        scratch_types=[
            pltpu.VMEM((2, chunk), jnp.int32),
            pltpu.VMEM((2, chunk, d), jnp.float32),
            pltpu.SemaphoreType.DMA,
            pltpu.SemaphoreType.DMA,
        ],
    )
    def gather_k(idx_hbm, table_hbm, out_hbm, idx_v, rows_v, sem0, sem1):
        wid = lax.axis_index("s") * nc + lax.axis_index("c")
        base = wid * per_w
        sems = (sem0, sem1)
        pltpu.sync_copy(idx_hbm.at[pl.ds(base, chunk)], idx_v.at[0])
        cp0 = pltpu.async_copy(table_hbm.at[idx_v.at[0]], rows_v.at[0], sems[0])
        copies = [cp0, None]
        for c in range(n_chunks):
            cur = c % 2
            nxt = (c + 1) % 2
            if c + 1 < n_chunks:
                off = base + (c + 1) * chunk
                pltpu.sync_copy(idx_hbm.at[pl.ds(off, chunk)], idx_v.at[nxt])
                copies[nxt] = pltpu.async_copy(
                    table_hbm.at[idx_v.at[nxt]], rows_v.at[nxt], sems[nxt])
            copies[cur].wait()
            pltpu.sync_copy(rows_v.at[cur],
                            out_hbm.at[pl.ds(base + c * chunk, chunk)])

    return gather_k(idx_flat, table)


def kernel(z_e, codebook):
    b, c, h, w = z_e.shape
    z = jnp.transpose(z_e, (0, 2, 3, 1)).reshape(-1, c)
    zsq = jnp.sum(z * z, axis=1, keepdims=True)
    csq = jnp.sum(codebook * codebook, axis=1)[None, :]
    idx2d = _nearest_idx_call(z_e.reshape(b, c, h * w), codebook, zsq, csq, _TM)
    idx = idx2d.reshape(-1)
    zq_flat = _gather_rows_call(codebook, idx)
    z_q = jnp.transpose(zq_flat.reshape(b, h, w, c), (0, 3, 1, 2))
    return z_q, idx.reshape(b, h, w)


# speedup vs baseline: 1.2106x; 1.0852x over previous
import functools

import jax
import jax.numpy as jnp
from jax import lax
from jax.experimental import pallas as pl
from jax.experimental.pallas import tpu as pltpu
from jax.experimental.pallas import tpu_sc as plsc

_TM = 2048


_WIN_EDGES = (0, 2736, 5472, 8192)
_CHUNKS = ((0, 1368), (1368, 2736), (2736, 4104), (4104, 5472),
           (5472, 6840), (6840, 8192))


def _argmin_body(z_ref, cb_ref, zsq_ref, csq_ref, idx_ref):
    z2 = z_ref[...] + z_ref[...]
    zsq = zsq_ref[...]
    tm = z2.shape[0]
    big = jnp.float32(2.0**30)
    col = lax.broadcasted_iota(jnp.int32, (tm, 1368), 1).astype(jnp.float32)

    acc_v = acc_i = None
    wv = wi = None
    for ch, (r0, r1) in enumerate(_CHUNKS):
        w = r1 - r0
        zc2 = lax.dot_general(
            z2, cb_ref[r0:r1, :], (((1,), (1,)), ((), ())),
            preferred_element_type=jnp.float32)
        d = (zsq - zc2) + csq_ref[:, r0:r1]
        lm = jnp.min(d, axis=1, keepdims=True)
        li = jnp.min(jnp.where(d == lm, col[:, :w], big),
                     axis=1, keepdims=True) + jnp.float32(r0)
        if ch % 2 == 0:
            wv, wi = lm, li
        else:
            better = lm < wv
            wi = jnp.where(better, li, wi)
            wv = jnp.where(better, lm, wv)
            if acc_v is None:
                acc_v, acc_i = wv, wi
            else:
                better = wv < acc_v
                acc_i = jnp.where(better, wi, acc_i)
                acc_v = jnp.where(better, wv, acc_v)
            acc_v = acc_v.astype(jnp.bfloat16).astype(jnp.float32)
    idx_ref[...] = acc_i.astype(jnp.int32)


def _nearest_idx_call(z, cb, zsq, csq, tm):
    n, c = z.shape
    k = cb.shape[0]
    grid = (n // tm,)
    return pl.pallas_call(
        _argmin_body,
        grid=grid,
        in_specs=[
            pl.BlockSpec((tm, c), lambda m: (m, 0)),
            pl.BlockSpec((k, c), lambda m: (0, 0)),
            pl.BlockSpec((tm, 1), lambda m: (m, 0)),
            pl.BlockSpec((1, k), lambda m: (0, 0)),
        ],
        out_specs=pl.BlockSpec((tm, 1), lambda m: (m, 0)),
        out_shape=jax.ShapeDtypeStruct((n, 1), jnp.int32),
        compiler_params=pltpu.CompilerParams(
            dimension_semantics=("arbitrary",)),
    )(z, cb, zsq, csq)


def _gather_rows_call(table, idx_flat):
    n = idx_flat.shape[0]
    d = table.shape[1]
    nc, ns = 2, 16
    nw = nc * ns
    chunk = 128
    per_w = n // nw
    n_chunks = per_w // chunk
    mesh = plsc.VectorSubcoreMesh(core_axis_name="c", subcore_axis_name="s")

    @functools.partial(
        pl.kernel,
        out_type=jax.ShapeDtypeStruct((n, d), jnp.float32),
        mesh=mesh,
        scratch_types=[
            pltpu.VMEM((2, chunk), jnp.int32),
            pltpu.VMEM((2, chunk, d), jnp.float32),
            pltpu.SemaphoreType.DMA,
            pltpu.SemaphoreType.DMA,
        ],
    )
    def gather_k(idx_hbm, table_hbm, out_hbm, idx_v, rows_v, sem0, sem1):
        wid = lax.axis_index("s") * nc + lax.axis_index("c")
        base = wid * per_w
        sems = (sem0, sem1)
        pltpu.sync_copy(idx_hbm.at[pl.ds(base, chunk)], idx_v.at[0])
        cp0 = pltpu.async_copy(table_hbm.at[idx_v.at[0]], rows_v.at[0], sems[0])
        copies = [cp0, None]
        for c in range(n_chunks):
            cur = c % 2
            nxt = (c + 1) % 2
            if c + 1 < n_chunks:
                off = base + (c + 1) * chunk
                pltpu.sync_copy(idx_hbm.at[pl.ds(off, chunk)], idx_v.at[nxt])
                copies[nxt] = pltpu.async_copy(
                    table_hbm.at[idx_v.at[nxt]], rows_v.at[nxt], sems[nxt])
            copies[cur].wait()
            pltpu.sync_copy(rows_v.at[cur],
                            out_hbm.at[pl.ds(base + c * chunk, chunk)])

    return gather_k(idx_flat, table)


def kernel(z_e, codebook):
    b, c, h, w = z_e.shape
    z = jnp.transpose(z_e, (0, 2, 3, 1)).reshape(-1, c)
    zsq = jnp.sum(z * z, axis=1, keepdims=True)
    csq = jnp.sum(codebook * codebook, axis=1)[None, :]
    idx2d = _nearest_idx_call(z, codebook, zsq, csq, _TM)
    idx = idx2d.reshape(-1)
    zq_flat = _gather_rows_call(codebook, idx)
    z_q = jnp.transpose(zq_flat.reshape(b, h, w, c), (0, 3, 1, 2))
    return z_q, idx.reshape(b, h, w)
